# baseline (speedup 1.0000x reference)
import functools

import jax
import jax.numpy as jnp
from jax import lax
from jax.experimental import pallas as pl
from jax.experimental.pallas import tpu as pltpu
from jax.experimental.pallas import tpu_sc as plsc

N = 262144
M = 64
K = 16

N_SC = 49152
N_TC = N - N_SC

NC = 2
NS = 16
NW = NC * NS
CHUNK2 = 256
ROW_UNROLL = 1

N_SC2 = N_SC // 2
SC_ROWS2_PER_W = N_SC2 // NW
SC_NCHUNK = SC_ROWS2_PER_W // CHUNK2

_GATHER_DNUMS = lax.GatherDimensionNumbers(
    offset_dims=(), collapsed_slice_dims=(0,), start_index_map=(0,))


def _lane_shuffle(v, idx):
    return lax.gather(v, idx[:, None], _GATHER_DNUMS, (1,),
                      mode=lax.GatherScatterMode.PROMISE_IN_BOUNDS)


def _lane_allreduce_sum(v):
    lane = lax.iota(jnp.int32, 16)
    for k in range(4):
        v = v + _lane_shuffle(v, lane ^ (1 << k))
    return v


def _row_compute(lbuf, obuf, r):
    p = [jnp.exp(lbuf[r, pl.ds(16 * j, 16)]) for j in range(8)]
    kh = None
    for _ in range(K):
        rinv = []
        for h in (0, 4):
            sv = _lane_allreduce_sum((p[h] + p[h + 1]) + (p[h + 2] + p[h + 3]))
            rinv.append(1.0 / sv)
        o = [x * rinv[j // 4] for j, x in enumerate(p)]
        kh = o if kh is None else [a + b for a, b in zip(kh, o)]
        p = [x * (1.0 - oo) for x, oo in zip(p, o)]
    for j in range(8):
        obuf[r, pl.ds(16 * j, 16)] = kh[j]


def _tec_body(l_hbm, out_hbm, lbuf, obuf):
    wid = lax.axis_index("s") * NC + lax.axis_index("c")
    base = wid * SC_ROWS2_PER_W

    def chunk_body(g, carry):
        start = base + g * CHUNK2
        pltpu.sync_copy(l_hbm.at[pl.ds(start, CHUNK2)], lbuf)

        def row_body(r):
            _row_compute(lbuf, obuf, r)

        plsc.parallel_loop(0, CHUNK2, 1, unroll=ROW_UNROLL)(row_body)
        pltpu.sync_copy(obuf, out_hbm.at[pl.ds(start, CHUNK2)])
        return carry

    lax.fori_loop(0, SC_NCHUNK, chunk_body, 0)


def _sc_part(l2):
    mesh = plsc.VectorSubcoreMesh(core_axis_name="c", subcore_axis_name="s",
                                  num_cores=NC, num_subcores=NS)
    return pl.kernel(
        _tec_body,
        out_type=jax.ShapeDtypeStruct((N_SC2, 2 * M), jnp.float32),
        mesh=mesh,
        scratch_types=[
            pltpu.VMEM((CHUNK2, 2 * M), jnp.float32),
            pltpu.VMEM((CHUNK2, 2 * M), jnp.float32),
        ],
    )(l2)


PRE_BLK = 512
PRE_NBLK = N_SC2 // PRE_BLK


def _pre_body(lt_ref, gt_ref, lb_ref, gb_ref, o_ref):
    o_ref[:, pl.ds(0, M)] = jnp.transpose(lt_ref[...] + gt_ref[...])
    o_ref[:, pl.ds(M, M)] = jnp.transpose(lb_ref[...] + gb_ref[...])


def _pre_part(lT, gT):
    return pl.pallas_call(
        _pre_body,
        out_shape=jax.ShapeDtypeStruct((N_SC2, 2 * M), jnp.float32),
        grid=(PRE_NBLK,),
        in_specs=[pl.BlockSpec((M, PRE_BLK), lambda i: (0, i)),
                  pl.BlockSpec((M, PRE_BLK), lambda i: (0, i)),
                  pl.BlockSpec((M, PRE_BLK), lambda i: (0, i + PRE_NBLK)),
                  pl.BlockSpec((M, PRE_BLK), lambda i: (0, i + PRE_NBLK))],
        out_specs=pl.BlockSpec((PRE_BLK, 2 * M), lambda i: (i, 0)),
    )(lT, gT, lT, gT)


TC_BLK = 2048
TC_BLK0 = N_SC // TC_BLK


def _tc_body(l_ref, g_ref, o_ref):
    p = jnp.exp(l_ref[...] + g_ref[...])
    kh = jnp.zeros_like(p)
    for _ in range(K):
        s = jnp.sum(p, axis=0, keepdims=True)
        r = 1.0 / s
        o = p * r
        kh = kh + o
        p = p * (1.0 - o)
    o_ref[...] = kh


def _tc_part(lT, gT):
    return pl.pallas_call(
        _tc_body,
        out_shape=jax.ShapeDtypeStruct((M, N), jnp.float32),
        grid=(N_TC // TC_BLK,),
        in_specs=[pl.BlockSpec((M, TC_BLK), lambda i: (0, i + TC_BLK0)),
                  pl.BlockSpec((M, TC_BLK), lambda i: (0, i + TC_BLK0))],
        out_specs=pl.BlockSpec((M, TC_BLK), lambda i: (0, i + TC_BLK0)),
    )(lT, gT)


@jax.jit
def _gtnm(logits, gumbel):
    lT = jnp.transpose(logits)
    gT = jnp.transpose(gumbel)
    l2 = _pre_part(lT, gT)
    sc_out = _sc_part(l2)
    tc_outT = _tc_part(lT, gT)
    scT = jnp.transpose(sc_out.reshape(N_SC2, 2, M), (2, 1, 0)).reshape(M, N_SC)
    outT = lax.dynamic_update_slice(tc_outT, scT, (0, 0))
    return jnp.transpose(outT)


def kernel(logits, gumbel):
    return _gtnm(logits, gumbel)


# speedup vs baseline: 1.5959x; 1.5959x over previous
import functools

import jax
import jax.numpy as jnp
from jax import lax
from jax.experimental import pallas as pl
from jax.experimental.pallas import tpu as pltpu
from jax.experimental.pallas import tpu_sc as plsc

N = 262144
M = 64
K = 16

N_SC = 49152
N_TC = N - N_SC

NC = 2
NS = 16
NW = NC * NS
CHUNK2 = 256
ROW_UNROLL = 1

N_SC2 = N_SC // 2
SC_ROWS2_PER_W = N_SC2 // NW
SC_NCHUNK = SC_ROWS2_PER_W // CHUNK2

_GATHER_DNUMS = lax.GatherDimensionNumbers(
    offset_dims=(), collapsed_slice_dims=(0,), start_index_map=(0,))


def _lane_shuffle(v, idx):
    return lax.gather(v, idx[:, None], _GATHER_DNUMS, (1,),
                      mode=lax.GatherScatterMode.PROMISE_IN_BOUNDS)


def _lane_allreduce_sum(v):
    lane = lax.iota(jnp.int32, 16)
    for k in range(4):
        v = v + _lane_shuffle(v, lane ^ (1 << k))
    return v


def _row_compute(lbuf, obuf, r):
    p = [jnp.exp(lbuf[r, pl.ds(16 * j, 16)]) for j in range(8)]
    kh = None
    for _ in range(K):
        rinv = []
        for h in (0, 4):
            sv = _lane_allreduce_sum((p[h] + p[h + 1]) + (p[h + 2] + p[h + 3]))
            rinv.append(1.0 / sv)
        o = [x * rinv[j // 4] for j, x in enumerate(p)]
        kh = o if kh is None else [a + b for a, b in zip(kh, o)]
        p = [x * (1.0 - oo) for x, oo in zip(p, o)]
    for j in range(8):
        obuf[r, pl.ds(16 * j, 16)] = kh[j]


def _tec_body(l_hbm, out_hbm, lbuf, obuf):
    wid = lax.axis_index("s") * NC + lax.axis_index("c")
    base = wid * SC_ROWS2_PER_W

    def chunk_body(g, carry):
        start = base + g * CHUNK2
        pltpu.sync_copy(l_hbm.at[pl.ds(start, CHUNK2)], lbuf)

        def row_body(r):
            _row_compute(lbuf, obuf, r)

        plsc.parallel_loop(0, CHUNK2, 1, unroll=ROW_UNROLL)(row_body)
        pltpu.sync_copy(obuf, out_hbm.at[pl.ds(start, CHUNK2)])
        return carry

    lax.fori_loop(0, SC_NCHUNK, chunk_body, 0)


def _sc_part(l2):
    mesh = plsc.VectorSubcoreMesh(core_axis_name="c", subcore_axis_name="s",
                                  num_cores=NC, num_subcores=NS)
    return pl.kernel(
        _tec_body,
        out_type=jax.ShapeDtypeStruct((N_SC2, 2 * M), jnp.float32),
        mesh=mesh,
        scratch_types=[
            pltpu.VMEM((CHUNK2, 2 * M), jnp.float32),
            pltpu.VMEM((CHUNK2, 2 * M), jnp.float32),
        ],
    )(l2)


PRE_BLK = 512
PRE_NBLK = N_SC2 // PRE_BLK


def _pre_body(lt_ref, gt_ref, lb_ref, gb_ref, o_ref):
    o_ref[:, pl.ds(0, M)] = jnp.transpose(lt_ref[...] + gt_ref[...])
    o_ref[:, pl.ds(M, M)] = jnp.transpose(lb_ref[...] + gb_ref[...])


def _pre_part(lT, gT):
    return pl.pallas_call(
        _pre_body,
        out_shape=jax.ShapeDtypeStruct((N_SC2, 2 * M), jnp.float32),
        grid=(PRE_NBLK,),
        in_specs=[pl.BlockSpec((M, PRE_BLK), lambda i: (0, i)),
                  pl.BlockSpec((M, PRE_BLK), lambda i: (0, i)),
                  pl.BlockSpec((M, PRE_BLK), lambda i: (0, i + PRE_NBLK)),
                  pl.BlockSpec((M, PRE_BLK), lambda i: (0, i + PRE_NBLK))],
        out_specs=pl.BlockSpec((PRE_BLK, 2 * M), lambda i: (i, 0)),
    )(lT, gT, lT, gT)


TC_BLK = 2048
TC_BLK0 = N_SC // TC_BLK


def _tc_body(l_ref, g_ref, o_ref):
    p = jnp.exp(l_ref[...] + g_ref[...])
    kh = jnp.zeros_like(p)
    for _ in range(K):
        s = jnp.sum(p, axis=0, keepdims=True)
        r = 1.0 / s
        o = p * r
        kh = kh + o
        p = p * (1.0 - o)
    o_ref[...] = kh


def _tc_part(lT, gT):
    return pl.pallas_call(
        _tc_body,
        out_shape=jax.ShapeDtypeStruct((M, N), jnp.float32),
        grid=(N_TC // TC_BLK,),
        in_specs=[pl.BlockSpec((M, TC_BLK), lambda i: (0, i + TC_BLK0)),
                  pl.BlockSpec((M, TC_BLK), lambda i: (0, i + TC_BLK0))],
        out_specs=pl.BlockSpec((M, TC_BLK), lambda i: (0, i + TC_BLK0)),
    )(lT, gT)


@jax.jit
def _gtnm(logits, gumbel):
    lT = jnp.transpose(logits)
    gT = jnp.transpose(gumbel)
    l2 = _pre_part(lT, gT)
    sc_out = _sc_part(l2)
    tc_outT = _tc_part(lT, gT)
    top = jnp.transpose(lax.slice(sc_out, (0, 0), (N_SC2, M)))
    bot = jnp.transpose(lax.slice(sc_out, (0, M), (N_SC2, 2 * M)))
    outT = lax.dynamic_update_slice(tc_outT, top, (0, 0))
    outT = lax.dynamic_update_slice(outT, bot, (0, N_SC2))
    return jnp.transpose(outT)


def kernel(logits, gumbel):
    return _gtnm(logits, gumbel)


# speedup vs baseline: 1.6146x; 1.0117x over previous
import functools

import jax
import jax.numpy as jnp
from jax import lax
from jax.experimental import pallas as pl
from jax.experimental.pallas import tpu as pltpu
from jax.experimental.pallas import tpu_sc as plsc

N = 262144
M = 64
K = 16

N_SC = 49152
N_TC = N - N_SC

NC = 2
NS = 16
NW = NC * NS
CHUNK2 = 256
ROW_UNROLL = 1

N_SC2 = N_SC // 2
SC_ROWS2_PER_W = N_SC2 // NW
SC_NCHUNK = SC_ROWS2_PER_W // CHUNK2

_GATHER_DNUMS = lax.GatherDimensionNumbers(
    offset_dims=(), collapsed_slice_dims=(0,), start_index_map=(0,))


def _lane_shuffle(v, idx):
    return lax.gather(v, idx[:, None], _GATHER_DNUMS, (1,),
                      mode=lax.GatherScatterMode.PROMISE_IN_BOUNDS)


def _lane_allreduce_sum(v):
    lane = lax.iota(jnp.int32, 16)
    for k in range(4):
        v = v + _lane_shuffle(v, lane ^ (1 << k))
    return v


def _row_compute(lbuf, obuf, r):
    p = [jnp.exp(lbuf[r, pl.ds(16 * j, 16)]) for j in range(8)]
    kh = None
    for _ in range(K):
        rinv = []
        for h in (0, 4):
            sv = _lane_allreduce_sum((p[h] + p[h + 1]) + (p[h + 2] + p[h + 3]))
            rinv.append(1.0 / sv)
        o = [x * rinv[j // 4] for j, x in enumerate(p)]
        kh = o if kh is None else [a + b for a, b in zip(kh, o)]
        p = [x * (1.0 - oo) for x, oo in zip(p, o)]
    for j in range(8):
        obuf[r, pl.ds(16 * j, 16)] = kh[j]


def _tec_body(l_hbm, out_hbm, lbuf, obuf):
    wid = lax.axis_index("s") * NC + lax.axis_index("c")
    base = wid * SC_ROWS2_PER_W

    def chunk_body(g, carry):
        start = base + g * CHUNK2
        pltpu.sync_copy(l_hbm.at[pl.ds(start, CHUNK2)], lbuf)

        def row_body(r):
            _row_compute(lbuf, obuf, r)

        plsc.parallel_loop(0, CHUNK2, 1, unroll=ROW_UNROLL)(row_body)
        pltpu.sync_copy(obuf, out_hbm.at[pl.ds(start, CHUNK2)])
        return carry

    lax.fori_loop(0, SC_NCHUNK, chunk_body, 0)


def _sc_part(l2):
    mesh = plsc.VectorSubcoreMesh(core_axis_name="c", subcore_axis_name="s",
                                  num_cores=NC, num_subcores=NS)
    return pl.kernel(
        _tec_body,
        out_type=jax.ShapeDtypeStruct((N_SC2, 2 * M), jnp.float32),
        mesh=mesh,
        scratch_types=[
            pltpu.VMEM((CHUNK2, 2 * M), jnp.float32),
            pltpu.VMEM((CHUNK2, 2 * M), jnp.float32),
        ],
    )(l2)


PRE_BLK = 512
PRE_NBLK = N_SC2 // PRE_BLK


def _pre_body(lt_ref, gt_ref, lb_ref, gb_ref, o_ref):
    o_ref[:, pl.ds(0, M)] = jnp.transpose(lt_ref[...] + gt_ref[...])
    o_ref[:, pl.ds(M, M)] = jnp.transpose(lb_ref[...] + gb_ref[...])


def _pre_part(lT, gT):
    return pl.pallas_call(
        _pre_body,
        out_shape=jax.ShapeDtypeStruct((N_SC2, 2 * M), jnp.float32),
        grid=(PRE_NBLK,),
        in_specs=[pl.BlockSpec((M, PRE_BLK), lambda i: (0, i)),
                  pl.BlockSpec((M, PRE_BLK), lambda i: (0, i)),
                  pl.BlockSpec((M, PRE_BLK), lambda i: (0, i + PRE_NBLK)),
                  pl.BlockSpec((M, PRE_BLK), lambda i: (0, i + PRE_NBLK))],
        out_specs=pl.BlockSpec((PRE_BLK, 2 * M), lambda i: (i, 0)),
    )(lT, gT, lT, gT)


TC_BLK = 4096
TC_BLK0 = N_SC // TC_BLK


def _tc_body(l_ref, g_ref, o_ref):
    p = jnp.exp(l_ref[...] + g_ref[...])
    kh = jnp.zeros_like(p)
    for _ in range(K):
        s = jnp.sum(p, axis=0, keepdims=True)
        r = 1.0 / s
        o = p * r
        kh = kh + o
        p = p * (1.0 - o)
    o_ref[...] = kh


def _tc_part(lT, gT):
    return pl.pallas_call(
        _tc_body,
        out_shape=jax.ShapeDtypeStruct((M, N), jnp.float32),
        grid=(N_TC // TC_BLK,),
        in_specs=[pl.BlockSpec((M, TC_BLK), lambda i: (0, i + TC_BLK0)),
                  pl.BlockSpec((M, TC_BLK), lambda i: (0, i + TC_BLK0))],
        out_specs=pl.BlockSpec((M, TC_BLK), lambda i: (0, i + TC_BLK0)),
    )(lT, gT)


@jax.jit
def _gtnm(logits, gumbel):
    lT = jnp.transpose(logits)
    gT = jnp.transpose(gumbel)
    l2 = _pre_part(lT, gT)
    sc_out = _sc_part(l2)
    tc_outT = _tc_part(lT, gT)
    top = jnp.transpose(lax.slice(sc_out, (0, 0), (N_SC2, M)))
    bot = jnp.transpose(lax.slice(sc_out, (0, M), (N_SC2, 2 * M)))
    outT = lax.dynamic_update_slice(tc_outT, top, (0, 0))
    outT = lax.dynamic_update_slice(outT, bot, (0, N_SC2))
    return jnp.transpose(outT)


def kernel(logits, gumbel):
    return _gtnm(logits, gumbel)
